# baseline (speedup 1.0000x reference)
import jax
import jax.numpy as jnp
from jax.experimental import pallas as pl
from jax.experimental.pallas import tpu as pltpu
from jax.experimental.pallas import tpu_sc as plsc

_D = 256
_B_BLK = 4


def kernel(ids, W):
    b, t = ids.shape
    idx = ids.astype(jnp.int32)
    mesh = plsc.VectorSubcoreMesh(core_axis_name="core",
                                  subcore_axis_name="subcore")

    @pl.kernel(
        out_type=jax.ShapeDtypeStruct((b, t, _D), jnp.float32),
        mesh=mesh,
        compiler_params=pltpu.CompilerParams(use_tc_tiling_on_sc=False),
    )
    def k(w_hbm, i_hbm, o_hbm):
        def body(i_vmem, o_vmem):
            for r in range(_B_BLK):
                pltpu.sync_copy(w_hbm.at[i_vmem.at[r]], o_vmem.at[r])

        pltpu.emit_pipeline(
            body,
            grid=(b // _B_BLK,),
            in_specs=[pl.BlockSpec((_B_BLK, t), index_map=lambda i: (i, 0))],
            out_specs=[pl.BlockSpec((_B_BLK, t, _D),
                                    index_map=lambda i: (i, 0, 0))],
            core_axis_name=("core", "subcore"),
            dimension_semantics=(pltpu.PARALLEL,),
        )(i_hbm, o_hbm)

    return k(W, idx)


# speedup vs baseline: 2.1595x; 2.1595x over previous
import jax
import jax.numpy as jnp
from jax.experimental import pallas as pl
from jax.experimental.pallas import tpu as pltpu
from jax.experimental.pallas import tpu_sc as plsc

_D = 256
_WINDOW = 128


def _sc_gather_t(W, idx_t):
    t, b = idx_t.shape
    mesh = plsc.VectorSubcoreMesh(core_axis_name="core",
                                  subcore_axis_name="subcore")

    @pl.kernel(
        out_type=jax.ShapeDtypeStruct((t, b, _D), jnp.float32),
        mesh=mesh,
    )
    def k(w_hbm, i_hbm, o_hbm):
        def body(i_vmem, o_vmem):
            pltpu.sync_copy(w_hbm.at[i_vmem.at[0]], o_vmem.at[0])

        pltpu.emit_pipeline(
            body,
            grid=(t, b // _WINDOW),
            in_specs=[pl.BlockSpec((1, _WINDOW),
                                   index_map=lambda i, j: (i, j))],
            out_specs=[pl.BlockSpec((1, _WINDOW, _D),
                                    index_map=lambda i, j: (i, j, 0))],
            core_axis_name=("core", "subcore"),
            dimension_semantics=(pltpu.PARALLEL, pltpu.PARALLEL),
        )(i_hbm, o_hbm)

    return k(W, idx_t)


def kernel(ids, W):
    idx_t = ids.astype(jnp.int32).T
    out_t = _sc_gather_t(W, idx_t)
    return out_t.transpose(1, 0, 2)
